# baseline (speedup 1.0000x reference)
import functools

import jax
import jax.numpy as jnp
from jax import lax
from jax.experimental import pallas as pl
from jax.experimental.pallas import tpu as pltpu
from jax.experimental.pallas import tpu_sc as plsc

N = 4096
DIN = 384
D = 512
DH = 256
K = 6
SCALE = D ** (-0.5)
BLK = 128
NBLK = N // BLK
NEG = float("-inf")

_PREC = lax.Precision.DEFAULT


def _dot(a, b):
    return lax.dot_general(a, b, (((1,), (0,)), ((), ())),
                           precision=_PREC, preferred_element_type=jnp.float32)


def _dot_t(a, b):
    return lax.dot_general(a, b, (((1,), (1,)), ((), ())),
                           precision=lax.Precision.DEFAULT,
                           preferred_element_type=jnp.float32)


def _leaky(x):
    return jnp.where(x >= 0, x, 0.01 * x)


def _k_abc(xp_ref, w1_ref, b1_ref, wh_ref, bh_ref, wt_ref, bt_ref,
           eh_ref, etb_ref, vals_ref, idx_ref, ehs, ets, cs):
    p = pl.program_id(0)
    i = pl.program_id(1)

    @pl.when(p == 0)
    def _():
        x1 = _leaky(_dot(xp_ref[...], w1_ref[...]) + b1_ref[...])

        @pl.when(i == 0)
        def _():
            cs[...] = jnp.zeros_like(cs)

        cs[...] += jnp.sum(x1, axis=0, keepdims=True)

    @pl.when(p == 1)
    def _():
        x1 = _leaky(_dot(xp_ref[...], w1_ref[...]) + b1_ref[...])
        x = (x1 + cs[...] * (1.0 / N)) * 0.5
        eh = _dot(x, wh_ref[...]) + bh_ref[...]
        et = _dot(x, wt_ref[...]) + bt_ref[...]
        eh_ref[...] = eh
        etb_ref[...] = et.astype(jnp.bfloat16)
        ehs[pl.ds(i * BLK, BLK), :] = eh
        ets[pl.ds(i * BLK, BLK), :] = et

    @pl.when(p == 2)
    def _():
        eh_ref[...] = ehs[pl.ds(0, BLK), :]
        etb_ref[...] = ets[pl.ds(0, BLK), :].astype(jnp.bfloat16)
        eh = ehs[pl.ds(i * BLK, BLK), :]
        logits = _dot_t(eh * SCALE, ets[...])
        cols = lax.broadcasted_iota(jnp.int32, (BLK, N), 1)
        kcol = lax.broadcasted_iota(jnp.int32, (BLK, K), 1)
        vals = jnp.full((BLK, K), NEG, jnp.float32)
        idxs = jnp.zeros((BLK, K), jnp.int32)
        x = logits
        for k in range(K):
            m = jnp.max(x, axis=1, keepdims=True)
            i_k = jnp.argmax(x, axis=1).astype(jnp.int32)[:, None]
            vals = jnp.where(kcol == k, m, vals)
            idxs = jnp.where(kcol == k, i_k, idxs)
            x = jnp.where(cols == i_k, NEG, x)
        vals_ref[...] = vals
        idx_ref[...] = idxs


def _abc(xp, w1, b1, wh, bh, wt, bt):
    return pl.pallas_call(
        _k_abc,
        grid=(3, NBLK),
        in_specs=[
            pl.BlockSpec((BLK, DIN), lambda p, i: (jnp.where(p == 2, 0, i), 0)),
            pl.BlockSpec((DIN, D), lambda p, i: (0, 0)),
            pl.BlockSpec((1, D), lambda p, i: (0, 0)),
            pl.BlockSpec((D, D), lambda p, i: (0, 0)),
            pl.BlockSpec((1, D), lambda p, i: (0, 0)),
            pl.BlockSpec((D, D), lambda p, i: (0, 0)),
            pl.BlockSpec((1, D), lambda p, i: (0, 0)),
        ],
        out_specs=[
            pl.BlockSpec((BLK, D), lambda p, i: (jnp.where(p == 1, i, 0), 0)),
            pl.BlockSpec((BLK, D), lambda p, i: (jnp.where(p == 1, i, 0), 0)),
            pl.BlockSpec((BLK, K), lambda p, i: (jnp.where(p == 2, i, 0), 0)),
            pl.BlockSpec((BLK, K), lambda p, i: (jnp.where(p == 2, i, 0), 0)),
        ],
        out_shape=[
            jax.ShapeDtypeStruct((N, D), jnp.float32),
            jax.ShapeDtypeStruct((N, D), jnp.bfloat16),
            jax.ShapeDtypeStruct((N, K), jnp.float32),
            jax.ShapeDtypeStruct((N, K), jnp.int32),
        ],
        scratch_shapes=[
            pltpu.VMEM((N, D), jnp.float32),
            pltpu.VMEM((N, D), jnp.float32),
            pltpu.VMEM((1, D), jnp.float32),
        ],
    )(xp, w1, b1, wh, bh, wt, bt)


_NW = 32
_PER_W = N * K // _NW
_NBUF = 4
_CH = 48
_NCH = _PER_W // _CH


_DG = D // 2


def _sc_gather(table, idx_flat):
    mesh = plsc.VectorSubcoreMesh(core_axis_name="c", subcore_axis_name="s")

    @functools.partial(
        pl.kernel,
        mesh=mesh,
        out_type=jax.ShapeDtypeStruct((N * K, _DG), jnp.int32),
        scratch_types=[
            pltpu.VMEM((_PER_W,), jnp.int32),
        ] + [pltpu.VMEM((_CH, _DG), jnp.int32)] * _NBUF
          + [pltpu.SemaphoreType.DMA] * (2 * _NBUF),
    )
    def k(table_hbm, idx_hbm, out_hbm, idx_v, *scr):
        bufs = scr[:_NBUF]
        gsem = scr[_NBUF:2 * _NBUF]
        wsem = scr[2 * _NBUF:]
        wid = lax.axis_index("s") * 2 + lax.axis_index("c")
        base = wid * _PER_W
        pltpu.sync_copy(idx_hbm.at[pl.ds(base, _PER_W)], idx_v)

        def gather(c):
            b = c % _NBUF
            return pltpu.async_copy(
                table_hbm.at[idx_v.at[pl.ds(c * _CH, _CH)]], bufs[b], gsem[b])

        def write(c):
            b = c % _NBUF
            return pltpu.async_copy(
                bufs[b], out_hbm.at[pl.ds(base + c * _CH, _CH)], wsem[b])

        gathers = [None] * _NCH
        writes = [None] * _NCH
        for c in range(_NBUF):
            gathers[c] = gather(c)
        for c in range(_NCH):
            gathers[c].wait()
            writes[c] = write(c)
            nc = c + _NBUF
            if nc < _NCH:
                writes[c].wait()
                gathers[nc] = gather(nc)
        for c in range(_NCH - _NBUF, _NCH):
            writes[c].wait()

    return k(table, idx_flat)


def _k_ef(eh_ref, vals_ref, nb_ref, wl1_ref, bl1_ref, wl2_ref, bl2_ref,
          wa1_ref, ba1_ref, wa2_ref, ba2_ref, out_ref, e2s, gls):
    p = pl.program_id(0)
    i = pl.program_id(1)

    @pl.when(p == 0)
    def _():
        h = eh_ref[...]
        v = vals_ref[...]
        kcol = lax.broadcasted_iota(jnp.int32, (BLK, K), 1)

        m = jnp.max(v, axis=1, keepdims=True)
        ev = jnp.exp(v - m)
        pr = ev / jnp.sum(ev, axis=1, keepdims=True)

        ka = jnp.full((BLK, K), NEG, jnp.float32)
        for k in range(K):
            nb_k = nb_ref[:, k, :].astype(jnp.float32)
            p_k = pr[:, k:k + 1]
            eh_r = p_k * nb_k + (1.0 - p_k) * h
            gate = jnp.tanh(h + eh_r)
            ka_k = (jnp.sum(nb_k, axis=1, keepdims=True)
                    * jnp.sum(gate, axis=1, keepdims=True))
            ka = jnp.where(kcol == k, ka_k, ka)

        m2 = jnp.max(ka, axis=1, keepdims=True)
        eka = jnp.exp(ka - m2)
        q = eka / jnp.sum(eka, axis=1, keepdims=True)

        e_nh = jnp.zeros((BLK, D), jnp.float32)
        for k in range(K):
            e_nh = e_nh + q[:, k:k + 1] * nb_ref[:, k, :].astype(jnp.float32)

        s_emb = _leaky(_dot(h + e_nh, wl1_ref[...]) + bl1_ref[...])
        b_emb = _leaky(_dot(h * e_nh, wl2_ref[...]) + bl2_ref[...])
        e2 = s_emb + b_emb
        e2s[pl.ds(i * BLK, BLK), :] = e2

        g1 = _leaky(_dot(e2, wa1_ref[...]) + ba1_ref[...])
        gls[pl.ds(i * BLK, BLK), :] = (
            jnp.sum(g1 * wa2_ref[...], axis=1, keepdims=True) + ba2_ref[...])

    @pl.when(p == 1)
    def _():
        gl = gls[...]
        m = jnp.max(gl, axis=0, keepdims=True)
        s = jnp.sum(jnp.exp(gl - m), axis=0, keepdims=True)
        gl_blk = gls[pl.ds(i * BLK, BLK), :]
        att = jnp.exp(gl_blk - m) / s

        @pl.when(i == 0)
        def _():
            out_ref[...] = jnp.zeros_like(out_ref)

        out_ref[...] += jnp.sum(att * e2s[pl.ds(i * BLK, BLK), :],
                                axis=0, keepdims=True)


def _ef(eh, vals, nb, wl1, bl1, wl2, bl2, wa1, ba1, wa2r, ba2):
    return pl.pallas_call(
        _k_ef,
        grid=(2, NBLK),
        in_specs=[
            pl.BlockSpec((BLK, D), lambda p, i: (jnp.where(p == 0, i, 0), 0)),
            pl.BlockSpec((BLK, K), lambda p, i: (jnp.where(p == 0, i, 0), 0)),
            pl.BlockSpec((BLK, K, D),
                         lambda p, i: (jnp.where(p == 0, i, 0), 0, 0)),
            pl.BlockSpec((D, D), lambda p, i: (0, 0)),
            pl.BlockSpec((1, D), lambda p, i: (0, 0)),
            pl.BlockSpec((D, D), lambda p, i: (0, 0)),
            pl.BlockSpec((1, D), lambda p, i: (0, 0)),
            pl.BlockSpec((D, DH), lambda p, i: (0, 0)),
            pl.BlockSpec((1, DH), lambda p, i: (0, 0)),
            pl.BlockSpec((1, DH), lambda p, i: (0, 0)),
            pl.BlockSpec((1, 1), lambda p, i: (0, 0)),
        ],
        out_specs=pl.BlockSpec((1, D), lambda p, i: (0, 0)),
        out_shape=jax.ShapeDtypeStruct((1, D), jnp.float32),
        scratch_shapes=[
            pltpu.VMEM((N, D), jnp.float32),
            pltpu.VMEM((N, 1), jnp.float32),
        ],
    )(eh, vals, nb, wl1, bl1, wl2, bl2, wa1, ba1, wa2r, ba2)


def kernel(x_path, W1, b1, Wh, bh, Wt, bt, Wl1, bl1, Wl2, bl2, Wa1, ba1, Wa2, ba2):
    xp = x_path.reshape(N, DIN)
    eh, etb, vals, idx = _abc(xp, W1, b1.reshape(1, D), Wh, bh.reshape(1, D),
                              Wt, bt.reshape(1, D))
    et32 = lax.bitcast_convert_type(etb.reshape(N, D // 2, 2), jnp.int32)
    nb32 = _sc_gather(et32, idx.reshape(N * K))
    nb = lax.bitcast_convert_type(nb32, jnp.bfloat16).reshape(N, K, D)
    return _ef(eh, vals, nb, Wl1, bl1.reshape(1, D), Wl2, bl2.reshape(1, D),
               Wa1, ba1.reshape(1, DH), Wa2.reshape(1, DH), ba2.reshape(1, 1))


# speedup vs baseline: 4.6959x; 4.6959x over previous
import functools

import jax
import jax.numpy as jnp
from jax import lax
from jax.experimental import pallas as pl
from jax.experimental.pallas import tpu as pltpu
from jax.experimental.pallas import tpu_sc as plsc

N = 4096
DIN = 384
D = 512
DH = 256
K = 6
SCALE = D ** (-0.5)
BLK = 128
NBLK = N // BLK
NEG = float("-inf")

_PREC = lax.Precision.DEFAULT


def _dot(a, b):
    return lax.dot_general(a, b, (((1,), (0,)), ((), ())),
                           precision=_PREC, preferred_element_type=jnp.float32)


def _dot_t(a, b):
    return lax.dot_general(a, b, (((1,), (1,)), ((), ())),
                           precision=lax.Precision.DEFAULT,
                           preferred_element_type=jnp.float32)


def _leaky(x):
    return jnp.where(x >= 0, x, 0.01 * x)


def _bf16_bits(x):
    u = lax.bitcast_convert_type(x, jnp.int32)
    return (u + 0x7FFF + ((u >> 16) & 1)) >> 16


def _pack_pair(x):
    lo = _bf16_bits(x[:, :D // 2])
    hi = _bf16_bits(x[:, D // 2:])
    return (lo & 0xFFFF) | (hi << 16)


def _unpack_pair(w):
    lo = lax.bitcast_convert_type(w << 16, jnp.float32)
    hi = lax.bitcast_convert_type((w >> 16) << 16, jnp.float32)
    return jnp.concatenate([lo, hi], axis=1)


def _k_abc(xp_ref, w1_ref, b1_ref, wh_ref, bh_ref, wt_ref, bt_ref,
           eh_ref, etb_ref, vals_ref, idx_ref, ehs, ets, cs):
    p = pl.program_id(0)
    i = pl.program_id(1)

    @pl.when(p == 0)
    def _():
        x1 = _leaky(_dot(xp_ref[...], w1_ref[...]) + b1_ref[...])

        @pl.when(i == 0)
        def _():
            cs[...] = jnp.zeros_like(cs)

        cs[...] += jnp.sum(x1, axis=0, keepdims=True)

    @pl.when(p == 1)
    def _():
        x1 = _leaky(_dot(xp_ref[...], w1_ref[...]) + b1_ref[...])
        x = (x1 + cs[...] * (1.0 / N)) * 0.5
        eh = _dot(x, wh_ref[...]) + bh_ref[...]
        et = _dot(x, wt_ref[...]) + bt_ref[...]
        eh_ref[...] = eh
        etb_ref[...] = _pack_pair(et)
        ehs[pl.ds(i * BLK, BLK), :] = eh
        ets[pl.ds(i * BLK, BLK), :] = et

    @pl.when(p == 2)
    def _():
        eh_ref[...] = ehs[pl.ds(0, BLK), :]
        etb_ref[...] = _pack_pair(ets[pl.ds(0, BLK), :])
        eh = ehs[pl.ds(i * BLK, BLK), :]
        logits = _dot_t(eh * SCALE, ets[...])
        cols = lax.broadcasted_iota(jnp.int32, (BLK, N), 1)
        kcol = lax.broadcasted_iota(jnp.int32, (BLK, K), 1)
        vals = jnp.full((BLK, K), NEG, jnp.float32)
        idxs = jnp.zeros((BLK, K), jnp.int32)
        x = logits
        for k in range(K):
            m = jnp.max(x, axis=1, keepdims=True)
            i_k = jnp.argmax(x, axis=1).astype(jnp.int32)[:, None]
            vals = jnp.where(kcol == k, m, vals)
            idxs = jnp.where(kcol == k, i_k, idxs)
            x = jnp.where(cols == i_k, NEG, x)
        vals_ref[...] = vals
        idx_ref[...] = idxs


def _abc(xp, w1, b1, wh, bh, wt, bt):
    return pl.pallas_call(
        _k_abc,
        grid=(3, NBLK),
        in_specs=[
            pl.BlockSpec((BLK, DIN), lambda p, i: (jnp.where(p == 2, 0, i), 0)),
            pl.BlockSpec((DIN, D), lambda p, i: (0, 0)),
            pl.BlockSpec((1, D), lambda p, i: (0, 0)),
            pl.BlockSpec((D, D), lambda p, i: (0, 0)),
            pl.BlockSpec((1, D), lambda p, i: (0, 0)),
            pl.BlockSpec((D, D), lambda p, i: (0, 0)),
            pl.BlockSpec((1, D), lambda p, i: (0, 0)),
        ],
        out_specs=[
            pl.BlockSpec((BLK, D), lambda p, i: (jnp.where(p == 1, i, 0), 0)),
            pl.BlockSpec((BLK, D // 2),
                         lambda p, i: (jnp.where(p == 1, i, 0), 0)),
            pl.BlockSpec((BLK, K), lambda p, i: (jnp.where(p == 2, i, 0), 0)),
            pl.BlockSpec((BLK, K), lambda p, i: (jnp.where(p == 2, i, 0), 0)),
        ],
        out_shape=[
            jax.ShapeDtypeStruct((N, D), jnp.float32),
            jax.ShapeDtypeStruct((N, D // 2), jnp.int32),
            jax.ShapeDtypeStruct((N, K), jnp.float32),
            jax.ShapeDtypeStruct((N, K), jnp.int32),
        ],
        scratch_shapes=[
            pltpu.VMEM((N, D), jnp.float32),
            pltpu.VMEM((N, D), jnp.float32),
            pltpu.VMEM((1, D), jnp.float32),
        ],
    )(xp, w1, b1, wh, bh, wt, bt)


_NW = 32
_PER_W = N * K // _NW
_NBUF = 4
_CH = 48
_NCH = _PER_W // _CH


_DG = D // 2


def _sc_gather(table, idx_flat):
    mesh = plsc.VectorSubcoreMesh(core_axis_name="c", subcore_axis_name="s")

    @functools.partial(
        pl.kernel,
        mesh=mesh,
        out_type=jax.ShapeDtypeStruct((N * K, _DG), jnp.int32),
        scratch_types=[
            pltpu.VMEM((_PER_W,), jnp.int32),
        ] + [pltpu.VMEM((_CH, _DG), jnp.int32)] * _NBUF
          + [pltpu.SemaphoreType.DMA] * (2 * _NBUF),
    )
    def k(table_hbm, idx_hbm, out_hbm, idx_v, *scr):
        bufs = scr[:_NBUF]
        gsem = scr[_NBUF:2 * _NBUF]
        wsem = scr[2 * _NBUF:]
        wid = lax.axis_index("s") * 2 + lax.axis_index("c")
        base = wid * _PER_W
        pltpu.sync_copy(idx_hbm.at[pl.ds(base, _PER_W)], idx_v)

        def gather(c):
            b = c % _NBUF
            return pltpu.async_copy(
                table_hbm.at[idx_v.at[pl.ds(c * _CH, _CH)]], bufs[b], gsem[b])

        def write(c):
            b = c % _NBUF
            return pltpu.async_copy(
                bufs[b], out_hbm.at[pl.ds(base + c * _CH, _CH)], wsem[b])

        gathers = [None] * _NCH
        writes = [None] * _NCH
        for c in range(_NBUF):
            gathers[c] = gather(c)
        for c in range(_NCH):
            gathers[c].wait()
            writes[c] = write(c)
            nc = c + _NBUF
            if nc < _NCH:
                writes[c].wait()
                gathers[nc] = gather(nc)
        for c in range(_NCH - _NBUF, _NCH):
            writes[c].wait()

    return k(table, idx_flat)


def _k_ef(eh_ref, vals_ref, nb_ref, wl1_ref, bl1_ref, wl2_ref, bl2_ref,
          wa1_ref, ba1_ref, wa2_ref, ba2_ref, out_ref, e2s, gls):
    p = pl.program_id(0)
    i = pl.program_id(1)

    @pl.when(p == 0)
    def _():
        h = eh_ref[...]
        v = vals_ref[...]
        kcol = lax.broadcasted_iota(jnp.int32, (BLK, K), 1)

        m = jnp.max(v, axis=1, keepdims=True)
        ev = jnp.exp(v - m)
        pr = ev / jnp.sum(ev, axis=1, keepdims=True)

        nbs = [_unpack_pair(nb_ref[:, k, :]) for k in range(K)]
        ka = jnp.full((BLK, K), NEG, jnp.float32)
        for k in range(K):
            nb_k = nbs[k]
            p_k = pr[:, k:k + 1]
            eh_r = p_k * nb_k + (1.0 - p_k) * h
            gate = jnp.tanh(h + eh_r)
            ka_k = (jnp.sum(nb_k, axis=1, keepdims=True)
                    * jnp.sum(gate, axis=1, keepdims=True))
            ka = jnp.where(kcol == k, ka_k, ka)

        m2 = jnp.max(ka, axis=1, keepdims=True)
        eka = jnp.exp(ka - m2)
        q = eka / jnp.sum(eka, axis=1, keepdims=True)

        e_nh = jnp.zeros((BLK, D), jnp.float32)
        for k in range(K):
            e_nh = e_nh + q[:, k:k + 1] * nbs[k]

        s_emb = _leaky(_dot(h + e_nh, wl1_ref[...]) + bl1_ref[...])
        b_emb = _leaky(_dot(h * e_nh, wl2_ref[...]) + bl2_ref[...])
        e2 = s_emb + b_emb
        e2s[pl.ds(i * BLK, BLK), :] = e2

        g1 = _leaky(_dot(e2, wa1_ref[...]) + ba1_ref[...])
        gls[pl.ds(i * BLK, BLK), :] = (
            jnp.sum(g1 * wa2_ref[...], axis=1, keepdims=True) + ba2_ref[...])

    @pl.when(p == 1)
    def _():
        gl = gls[...]
        m = jnp.max(gl, axis=0, keepdims=True)
        s = jnp.sum(jnp.exp(gl - m), axis=0, keepdims=True)
        gl_blk = gls[pl.ds(i * BLK, BLK), :]
        att = jnp.exp(gl_blk - m) / s

        @pl.when(i == 0)
        def _():
            out_ref[...] = jnp.zeros_like(out_ref)

        out_ref[...] += jnp.sum(att * e2s[pl.ds(i * BLK, BLK), :],
                                axis=0, keepdims=True)


def _ef(eh, vals, nb, wl1, bl1, wl2, bl2, wa1, ba1, wa2r, ba2):
    return pl.pallas_call(
        _k_ef,
        grid=(2, NBLK),
        in_specs=[
            pl.BlockSpec((BLK, D), lambda p, i: (jnp.where(p == 0, i, 0), 0)),
            pl.BlockSpec((BLK, K), lambda p, i: (jnp.where(p == 0, i, 0), 0)),
            pl.BlockSpec((BLK, K, D // 2),
                         lambda p, i: (jnp.where(p == 0, i, 0), 0, 0)),
            pl.BlockSpec((D, D), lambda p, i: (0, 0)),
            pl.BlockSpec((1, D), lambda p, i: (0, 0)),
            pl.BlockSpec((D, D), lambda p, i: (0, 0)),
            pl.BlockSpec((1, D), lambda p, i: (0, 0)),
            pl.BlockSpec((D, DH), lambda p, i: (0, 0)),
            pl.BlockSpec((1, DH), lambda p, i: (0, 0)),
            pl.BlockSpec((1, DH), lambda p, i: (0, 0)),
            pl.BlockSpec((1, 1), lambda p, i: (0, 0)),
        ],
        out_specs=pl.BlockSpec((1, D), lambda p, i: (0, 0)),
        out_shape=jax.ShapeDtypeStruct((1, D), jnp.float32),
        scratch_shapes=[
            pltpu.VMEM((N, D), jnp.float32),
            pltpu.VMEM((N, 1), jnp.float32),
        ],
    )(eh, vals, nb, wl1, bl1, wl2, bl2, wa1, ba1, wa2r, ba2)


def kernel(x_path, W1, b1, Wh, bh, Wt, bt, Wl1, bl1, Wl2, bl2, Wa1, ba1, Wa2, ba2):
    xp = x_path.reshape(N, DIN)
    eh, etb, vals, idx = _abc(xp, W1, b1.reshape(1, D), Wh, bh.reshape(1, D),
                              Wt, bt.reshape(1, D))
    nb = _sc_gather(etb, idx.reshape(N * K)).reshape(N, K, D // 2)
    return _ef(eh, vals, nb, Wl1, bl1.reshape(1, D), Wl2, bl2.reshape(1, D),
               Wa1, ba1.reshape(1, DH), Wa2.reshape(1, DH), ba2.reshape(1, 1))
